# speedup vs baseline: 10.0662x; 10.0662x over previous
import functools
import jax
import jax.numpy as jnp
from jax import lax
from jax.experimental import pallas as pl
from jax.experimental.pallas import tpu as pltpu
from jax.experimental.pallas import tpu_sc as plsc

N = 10000
D = 128
E = 320000

NC = 2
NS = 16
NW = NC * NS
CH = 128

NP = 10240
RPT = NP // NS
EPW = ((E // NW + CH - 1) // CH) * CH
EP = EPW * NW
NSTEP = EPW // CH

_mesh = plsc.VectorSubcoreMesh(core_axis_name="c", subcore_axis_name="s")


def _wid():
  return lax.axis_index("s") * NC + lax.axis_index("c")


def _zero_vmem_rows(ref, nrows):
  z = jnp.zeros((16,), jnp.float32)

  def body(r, carry):
    for c in range(D // 16):
      ref[r, pl.ds(c * 16, 16)] = z
    return carry

  lax.fori_loop(0, nrows, body, 0)


def _zero_vmem_1d(ref, n):
  z = jnp.zeros((16,), jnp.float32)

  def body(i, carry):
    ref[pl.ds(i * 16, 16)] = z
    return carry

  lax.fori_loop(0, n // 16, body, 0)


@functools.partial(
    pl.kernel,
    out_type=jax.ShapeDtypeStruct((NC, NP), jnp.float32),
    mesh=_mesh,
    scratch_types=dict(
        didx=pltpu.VMEM((CH,), jnp.int32),
        ones_v=pltpu.VMEM((CH,), jnp.float32),
        zv=pltpu.VMEM((RPT,), jnp.float32),
        acc=pltpu.VMEM_SHARED((NP,), jnp.float32),
    ),
)
def _deg_kernel(dst_hbm, out_hbm, didx, ones_v, zv, acc):
  cid = lax.axis_index("c")
  sid = lax.axis_index("s")
  wid = _wid()

  _zero_vmem_1d(zv, RPT)

  def setone(i, carry):
    ones_v[pl.ds(i * 16, 16)] = jnp.ones((16,), jnp.float32)
    return carry

  lax.fori_loop(0, CH // 16, setone, 0)

  pltpu.sync_copy(zv, acc.at[pl.ds(sid * RPT, RPT)])
  plsc.subcore_barrier()

  ebase = wid * EPW

  def step(j, carry):
    b = ebase + j * CH
    pltpu.sync_copy(dst_hbm.at[pl.ds(b, CH)], didx)
    pltpu.sync_copy(ones_v, acc.at[didx], add=True)
    return carry

  lax.fori_loop(0, NSTEP, step, 0)
  plsc.subcore_barrier()

  pltpu.sync_copy(acc.at[pl.ds(sid * RPT, RPT)],
                  out_hbm.at[cid, pl.ds(sid * RPT, RPT)])


@functools.partial(
    pl.kernel,
    out_type=jax.ShapeDtypeStruct((NC, NP, D), jnp.float32),
    mesh=_mesh,
    scratch_types=dict(
        sidx=pltpu.VMEM((CH,), jnp.int32),
        didx=pltpu.VMEM((CH,), jnp.int32),
        rows=pltpu.VMEM((CH, D), jnp.float32),
        acc=pltpu.VMEM_SHARED((NP, D), jnp.float32),
        sem=pltpu.SemaphoreType.DMA,
    ),
)
def _mp_kernel(y_hbm, src_hbm, dst_hbm, out_hbm, sidx, didx, rows, acc, sem):
  cid = lax.axis_index("c")
  sid = lax.axis_index("s")
  wid = _wid()

  _zero_vmem_rows(rows, CH)
  for k in range(RPT // CH):
    pltpu.sync_copy(rows, acc.at[pl.ds(sid * RPT + k * CH, CH)])
  plsc.subcore_barrier()

  ebase = wid * EPW

  def step(j, carry):
    b = ebase + j * CH
    pltpu.sync_copy(src_hbm.at[pl.ds(b, CH)], sidx)
    pltpu.async_copy(y_hbm.at[sidx], rows, sem).wait()
    pltpu.sync_copy(dst_hbm.at[pl.ds(b, CH)], didx)
    pltpu.sync_copy(rows, acc.at[didx], add=True)
    return carry

  lax.fori_loop(0, NSTEP, step, 0)
  plsc.subcore_barrier()

  pltpu.sync_copy(acc.at[pl.ds(sid * RPT, RPT)],
                  out_hbm.at[cid, pl.ds(sid * RPT, RPT)])


_B = 1024


def _tc_matmul_body(x_ref, w_ref, o_ref):
  o_ref[...] = jnp.dot(x_ref[...], w_ref[...],
                       preferred_element_type=jnp.float32)


def _tc_matmul(x, w):
  return pl.pallas_call(
      _tc_matmul_body,
      grid=(NP // _B,),
      in_specs=[
          pl.BlockSpec((_B, D), lambda i: (i, 0)),
          pl.BlockSpec((D, D), lambda i: (0, 0)),
      ],
      out_specs=pl.BlockSpec((_B, D), lambda i: (i, 0)),
      out_shape=jax.ShapeDtypeStruct((NP, D), jnp.float32),
  )(x, w)


def _tc_scale_body(deg_ref, xw_ref, y_ref, dinv_ref):
  deg = deg_ref[0, :] + deg_ref[1, :] + 1.0
  dinv = lax.rsqrt(deg)
  dinv_ref[...] = dinv
  y_ref[...] = dinv[:, None] * xw_ref[...]


def _tc_scale(deg2, xw):
  return pl.pallas_call(
      _tc_scale_body,
      grid=(NP // _B,),
      in_specs=[
          pl.BlockSpec((NC, _B), lambda i: (0, i)),
          pl.BlockSpec((_B, D), lambda i: (i, 0)),
      ],
      out_specs=[
          pl.BlockSpec((_B, D), lambda i: (i, 0)),
          pl.BlockSpec((_B,), lambda i: (i,)),
      ],
      out_shape=[
          jax.ShapeDtypeStruct((NP, D), jnp.float32),
          jax.ShapeDtypeStruct((NP,), jnp.float32),
      ],
  )(deg2, xw)


def _ln_relu(t, g, b, eps=1e-5):
  m = jnp.mean(t, axis=-1, keepdims=True)
  v = jnp.mean((t - m) * (t - m), axis=-1, keepdims=True)
  h = (t - m) * lax.rsqrt(v + eps) * g[None, :] + b[None, :]
  return jnp.maximum(h, 0.0)


def _tc_mid_body(s_ref, y_ref, dinv_ref, b_ref, g_ref, be_ref, w_ref, o_ref):
  t = s_ref[0] + s_ref[1] + y_ref[...]
  t = dinv_ref[...][:, None] * t + b_ref[...][None, :]
  h = _ln_relu(t, g_ref[...], be_ref[...])
  o_ref[...] = (dinv_ref[...][:, None]
                * jnp.dot(h, w_ref[...], preferred_element_type=jnp.float32))


def _tc_mid(s, y, dinv, b, g, be, w):
  return pl.pallas_call(
      _tc_mid_body,
      grid=(NP // _B,),
      in_specs=[
          pl.BlockSpec((NC, _B, D), lambda i: (0, i, 0)),
          pl.BlockSpec((_B, D), lambda i: (i, 0)),
          pl.BlockSpec((_B,), lambda i: (i,)),
          pl.BlockSpec((D,), lambda i: (0,)),
          pl.BlockSpec((D,), lambda i: (0,)),
          pl.BlockSpec((D,), lambda i: (0,)),
          pl.BlockSpec((D, D), lambda i: (0, 0)),
      ],
      out_specs=pl.BlockSpec((_B, D), lambda i: (i, 0)),
      out_shape=jax.ShapeDtypeStruct((NP, D), jnp.float32),
  )(s, y, dinv, b, g, be, w)


def _tc_final_body(s_ref, y_ref, dinv_ref, b_ref, g_ref, be_ref, o_ref):
  t = s_ref[0] + s_ref[1] + y_ref[...]
  t = dinv_ref[...][:, None] * t + b_ref[...][None, :]
  o_ref[...] = _ln_relu(t, g_ref[...], be_ref[...])


def _tc_final(s, y, dinv, b, g, be):
  return pl.pallas_call(
      _tc_final_body,
      grid=(NP // _B,),
      in_specs=[
          pl.BlockSpec((NC, _B, D), lambda i: (0, i, 0)),
          pl.BlockSpec((_B, D), lambda i: (i, 0)),
          pl.BlockSpec((_B,), lambda i: (i,)),
          pl.BlockSpec((D,), lambda i: (0,)),
          pl.BlockSpec((D,), lambda i: (0,)),
          pl.BlockSpec((D,), lambda i: (0,)),
      ],
      out_specs=pl.BlockSpec((_B, D), lambda i: (i, 0)),
      out_shape=jax.ShapeDtypeStruct((NP, D), jnp.float32),
  )(s, y, dinv, b, g, be)


def kernel(x, edge_index, W1, b1, g1, be1, W2, b2, g2, be2):
  pad_e = jnp.full((EP - E,), N, dtype=jnp.int32)
  src_p = jnp.concatenate([edge_index[0], pad_e])
  dst_p = jnp.concatenate([edge_index[1], pad_e])
  x_p = jnp.concatenate([x, jnp.zeros((NP - N, D), x.dtype)], axis=0)

  deg2 = _deg_kernel(dst_p)
  xw1 = _tc_matmul(x_p, W1)
  y1, dinv = _tc_scale(deg2, xw1)
  s1 = _mp_kernel(y1, src_p, dst_p)
  y2 = _tc_mid(s1, y1, dinv, b1, g1, be1, W2)
  s2 = _mp_kernel(y2, src_p, dst_p)
  out = _tc_final(s2, y2, dinv, b2, g2, be2)
  return out[:N]


# speedup vs baseline: 17.5527x; 1.7437x over previous
import functools
import jax
import jax.numpy as jnp
from jax import lax
from jax.experimental import pallas as pl
from jax.experimental.pallas import tpu as pltpu
from jax.experimental.pallas import tpu_sc as plsc

N = 10000
D = 128
E = 320000

NC = 2
NS = 16
NW = NC * NS
CH = 120

NB = 3
NBI = 6

NP = 10016
RPA = 624
TAIL = NP - NS * RPA
NPT = 10240
NPD = NPT
RPTD = NPD // NS
EPW = ((E // NW + NBI * CH - 1) // (NBI * CH)) * (NBI * CH)
EP = EPW * NW
NSTEP = EPW // CH
NGRP = NSTEP // NBI

_mesh = plsc.VectorSubcoreMesh(core_axis_name="c", subcore_axis_name="s")


def _wid():
  return lax.axis_index("s") * NC + lax.axis_index("c")


def _zero_vmem_rows(ref, nrows):
  z = jnp.zeros((16,), jnp.float32)

  def body(r, carry):
    for c in range(D // 16):
      ref[r, pl.ds(c * 16, 16)] = z
    return carry

  lax.fori_loop(0, nrows, body, 0)


def _zero_vmem_1d(ref, n):
  z = jnp.zeros((16,), jnp.float32)

  def body(i, carry):
    ref[pl.ds(i * 16, 16)] = z
    return carry

  lax.fori_loop(0, n // 16, body, 0)


@functools.partial(
    pl.kernel,
    out_type=jax.ShapeDtypeStruct((NC, NPD), jnp.float32),
    mesh=_mesh,
    scratch_types=dict(
        didx=pltpu.VMEM((NBI, CH), jnp.int32),
        ones_v=pltpu.VMEM((CH,), jnp.float32),
        zv=pltpu.VMEM((RPTD,), jnp.float32),
        acc=pltpu.VMEM_SHARED((NPD,), jnp.float32),
        isems=[pltpu.SemaphoreType.DMA for _ in range(NBI)],
        ssems=[pltpu.SemaphoreType.DMA for _ in range(NBI)],
    ),
)
def _deg_kernel(dst_hbm, out_hbm, didx, ones_v, zv, acc, isems, ssems):
  cid = lax.axis_index("c")
  sid = lax.axis_index("s")
  wid = _wid()

  _zero_vmem_1d(zv, RPTD)

  one = jnp.ones((16,), jnp.float32)
  for i in range(CH // 16):
    ones_v[pl.ds(i * 16, 16)] = one
  if CH % 16:
    ones_v[pl.ds(CH - 16, 16)] = one

  ebase = wid * EPW

  def fire_iload(j, ib):
    pltpu.async_copy(dst_hbm.at[pl.ds(ebase + j * CH, CH)], didx.at[ib],
                     isems[ib])

  def wait_iload(j, ib):
    pltpu.make_async_copy(dst_hbm.at[pl.ds(ebase + j * CH, CH)],
                          didx.at[ib], isems[ib]).wait()

  for ib in range(NBI):
    fire_iload(ib, ib)
  pltpu.sync_copy(zv, acc.at[pl.ds(sid * RPTD, RPTD)])
  plsc.subcore_barrier()

  def group(g, carry):
    for b6 in range(NBI):
      j = g * NBI + b6
      wait_iload(j, b6)
      pltpu.async_copy(ones_v, acc.at[didx.at[b6]], ssems[b6], add=True)
      pltpu.make_async_copy(ones_v, acc.at[didx.at[b6]], ssems[b6]).wait()

      @pl.when(j + NBI < NSTEP)
      def _():
        fire_iload(j + NBI, b6)

    return carry

  lax.fori_loop(0, NGRP, group, 0)
  plsc.subcore_barrier()

  pltpu.sync_copy(acc.at[pl.ds(sid * RPTD, RPTD)],
                  out_hbm.at[cid, pl.ds(sid * RPTD, RPTD)])


@functools.partial(
    pl.kernel,
    out_type=jax.ShapeDtypeStruct((NC, NPT, D), jnp.float32),
    mesh=_mesh,
    scratch_types=dict(
        sidx=pltpu.VMEM((NBI, CH), jnp.int32),
        didx=pltpu.VMEM((NBI, CH), jnp.int32),
        rows=pltpu.VMEM((NB, CH, D), jnp.float32),
        acc=pltpu.VMEM_SHARED((NP, D), jnp.float32),
        isems=[pltpu.SemaphoreType.DMA for _ in range(NBI)],
        gsems=[pltpu.SemaphoreType.DMA for _ in range(NB)],
        ssems=[pltpu.SemaphoreType.DMA for _ in range(NB)],
    ),
)
def _mp_kernel(y_hbm, src_hbm, dst_hbm, out_hbm, sidx, didx, rows, acc,
               isems, gsems, ssems):
  cid = lax.axis_index("c")
  sid = lax.axis_index("s")
  wid = _wid()
  ebase = wid * EPW

  def fire_iload(j, ib):
    pltpu.async_copy(src_hbm.at[pl.ds(ebase + j * CH, CH)], sidx.at[ib],
                     isems[ib])
    pltpu.async_copy(dst_hbm.at[pl.ds(ebase + j * CH, CH)], didx.at[ib],
                     isems[ib])

  def wait_iload(j, ib):
    pltpu.make_async_copy(src_hbm.at[pl.ds(ebase + j * CH, CH)],
                          sidx.at[ib], isems[ib]).wait()
    pltpu.make_async_copy(dst_hbm.at[pl.ds(ebase + j * CH, CH)],
                          didx.at[ib], isems[ib]).wait()

  def fire_gather(j, ib, b):
    pltpu.async_copy(y_hbm.at[sidx.at[ib]], rows.at[b], gsems[b])

  for ib in range(NBI):
    fire_iload(ib, ib)

  _zero_vmem_rows(rows.at[0], CH)
  for k in range(RPA // CH):
    pltpu.sync_copy(rows.at[0], acc.at[pl.ds(sid * RPA + k * CH, CH)])
  if RPA % CH:
    pltpu.sync_copy(rows.at[0].at[pl.ds(0, RPA % CH)],
                    acc.at[pl.ds(sid * RPA + (RPA // CH) * CH, RPA % CH)])

  @pl.when(sid == NS - 1)
  def _():
    pltpu.sync_copy(rows.at[0].at[pl.ds(0, TAIL)],
                    acc.at[pl.ds(NS * RPA, TAIL)])

  for b in range(NB):
    wait_iload(b, b)
    fire_gather(b, b, b)
  plsc.subcore_barrier()

  def group(g, carry):
    for b6 in range(NBI):
      j = g * NBI + b6
      b = b6 % NB
      pltpu.make_async_copy(y_hbm.at[sidx.at[b6]], rows.at[b],
                            gsems[b]).wait()
      pltpu.async_copy(rows.at[b], acc.at[didx.at[b6]], ssems[b], add=True)
      pltpu.make_async_copy(rows.at[b], acc.at[didx.at[b6]], ssems[b]).wait()

      @pl.when(j + NBI < NSTEP)
      def _():
        fire_iload(j + NBI, b6)

      @pl.when(j + NB < NSTEP)
      def _():
        ib2 = (b6 + NB) % NBI
        wait_iload(j + NB, ib2)
        fire_gather(j + NB, ib2, b)

    return carry

  lax.fori_loop(0, NGRP, group, 0)
  plsc.subcore_barrier()

  pltpu.sync_copy(acc.at[pl.ds(sid * RPA, RPA)],
                  out_hbm.at[cid, pl.ds(sid * RPA, RPA)])

  @pl.when(sid == NS - 1)
  def _():
    pltpu.sync_copy(acc.at[pl.ds(NS * RPA, TAIL)],
                    out_hbm.at[cid, pl.ds(NS * RPA, TAIL)])


_B = 1024


def _tc_matmul_body(x_ref, w_ref, o_ref):
  o_ref[...] = jnp.dot(x_ref[...], w_ref[...],
                       preferred_element_type=jnp.float32)


def _tc_matmul(x, w):
  return pl.pallas_call(
      _tc_matmul_body,
      grid=(NPT // _B,),
      in_specs=[
          pl.BlockSpec((_B, D), lambda i: (i, 0)),
          pl.BlockSpec((D, D), lambda i: (0, 0)),
      ],
      out_specs=pl.BlockSpec((_B, D), lambda i: (i, 0)),
      out_shape=jax.ShapeDtypeStruct((NPT, D), jnp.float32),
  )(x, w)


def _tc_scale_body(deg_ref, xw_ref, y_ref, dinv_ref):
  deg = deg_ref[0, :] + deg_ref[1, :] + 1.0
  dinv = lax.rsqrt(deg)
  dinv_ref[...] = dinv
  y_ref[...] = dinv[:, None] * xw_ref[...]


def _tc_scale(deg2, xw):
  return pl.pallas_call(
      _tc_scale_body,
      grid=(NPT // _B,),
      in_specs=[
          pl.BlockSpec((NC, _B), lambda i: (0, i)),
          pl.BlockSpec((_B, D), lambda i: (i, 0)),
      ],
      out_specs=[
          pl.BlockSpec((_B, D), lambda i: (i, 0)),
          pl.BlockSpec((_B,), lambda i: (i,)),
      ],
      out_shape=[
          jax.ShapeDtypeStruct((NPT, D), jnp.float32),
          jax.ShapeDtypeStruct((NPT,), jnp.float32),
      ],
  )(deg2, xw)


def _ln_relu(t, g, b, eps=1e-5):
  m = jnp.mean(t, axis=-1, keepdims=True)
  v = jnp.mean((t - m) * (t - m), axis=-1, keepdims=True)
  h = (t - m) * lax.rsqrt(v + eps) * g[None, :] + b[None, :]
  return jnp.maximum(h, 0.0)


def _tc_mid_body(s_ref, y_ref, dinv_ref, b_ref, g_ref, be_ref, w_ref, o_ref):
  t = s_ref[0] + s_ref[1] + y_ref[...]
  t = dinv_ref[...][:, None] * t + b_ref[...][None, :]
  h = _ln_relu(t, g_ref[...], be_ref[...])
  o_ref[...] = (dinv_ref[...][:, None]
                * jnp.dot(h, w_ref[...], preferred_element_type=jnp.float32))


def _tc_mid(s, y, dinv, b, g, be, w):
  return pl.pallas_call(
      _tc_mid_body,
      grid=(NPT // _B,),
      in_specs=[
          pl.BlockSpec((NC, _B, D), lambda i: (0, i, 0)),
          pl.BlockSpec((_B, D), lambda i: (i, 0)),
          pl.BlockSpec((_B,), lambda i: (i,)),
          pl.BlockSpec((D,), lambda i: (0,)),
          pl.BlockSpec((D,), lambda i: (0,)),
          pl.BlockSpec((D,), lambda i: (0,)),
          pl.BlockSpec((D, D), lambda i: (0, 0)),
      ],
      out_specs=pl.BlockSpec((_B, D), lambda i: (i, 0)),
      out_shape=jax.ShapeDtypeStruct((NPT, D), jnp.float32),
  )(s, y, dinv, b, g, be, w)


def _tc_final_body(s_ref, y_ref, dinv_ref, b_ref, g_ref, be_ref, o_ref):
  t = s_ref[0] + s_ref[1] + y_ref[...]
  t = dinv_ref[...][:, None] * t + b_ref[...][None, :]
  o_ref[...] = _ln_relu(t, g_ref[...], be_ref[...])


def _tc_final(s, y, dinv, b, g, be):
  return pl.pallas_call(
      _tc_final_body,
      grid=(NPT // _B,),
      in_specs=[
          pl.BlockSpec((NC, _B, D), lambda i: (0, i, 0)),
          pl.BlockSpec((_B, D), lambda i: (i, 0)),
          pl.BlockSpec((_B,), lambda i: (i,)),
          pl.BlockSpec((D,), lambda i: (0,)),
          pl.BlockSpec((D,), lambda i: (0,)),
          pl.BlockSpec((D,), lambda i: (0,)),
      ],
      out_specs=pl.BlockSpec((_B, D), lambda i: (i, 0)),
      out_shape=jax.ShapeDtypeStruct((NPT, D), jnp.float32),
  )(s, y, dinv, b, g, be)


def kernel(x, edge_index, W1, b1, g1, be1, W2, b2, g2, be2):
  pad_e = jnp.full((EP - E,), N, dtype=jnp.int32)
  src_p = jnp.concatenate([edge_index[0], pad_e])
  dst_p = jnp.concatenate([edge_index[1], pad_e])
  x_p = jnp.concatenate([x, jnp.zeros((NPT - N, D), x.dtype)], axis=0)

  deg2 = _deg_kernel(dst_p)
  xw1 = _tc_matmul(x_p, W1)
  y1, dinv = _tc_scale(deg2, xw1)
  s1 = _mp_kernel(y1, src_p, dst_p)
  y2 = _tc_mid(s1, y1, dinv, b1, g1, be1, W2)
  s2 = _mp_kernel(y2, src_p, dst_p)
  out = _tc_final(s2, y2, dinv, b2, g2, be2)
  return out[:N]


# speedup vs baseline: 19.7146x; 1.1232x over previous
import functools
import jax
import jax.numpy as jnp
from jax import lax
from jax.experimental import pallas as pl
from jax.experimental.pallas import tpu as pltpu
from jax.experimental.pallas import tpu_sc as plsc

N = 10000
D = 128
E = 320000

NC = 2
NS = 16
NW = NC * NS
CH = 120

NB = 3
NBI = 6

NP = 10016
RPA = 624
TAIL = NP - NS * RPA
NPT = 10240
NPD = NPT
RPTD = NPD // NS
EPW = ((E // NW + NBI * CH - 1) // (NBI * CH)) * (NBI * CH)
EP = EPW * NW
NSTEP = EPW // CH
NGRP = NSTEP // NBI

GA = 20
GB = 2 * NGRP - GA
EPWA = GA * NBI * CH
EPWB = GB * NBI * CH

_mesh = plsc.VectorSubcoreMesh(core_axis_name="c", subcore_axis_name="s")


def _wid():
  return lax.axis_index("s") * NC + lax.axis_index("c")


def _zero_vmem_rows(ref, nrows):
  z = jnp.zeros((16,), jnp.float32)

  def body(r, carry):
    for c in range(D // 16):
      ref[r, pl.ds(c * 16, 16)] = z
    return carry

  lax.fori_loop(0, nrows, body, 0)


def _zero_vmem_1d(ref, n):
  z = jnp.zeros((16,), jnp.float32)

  def body(i, carry):
    ref[pl.ds(i * 16, 16)] = z
    return carry

  lax.fori_loop(0, n // 16, body, 0)


@functools.partial(
    pl.kernel,
    out_type=jax.ShapeDtypeStruct((NC, NPD), jnp.float32),
    mesh=_mesh,
    scratch_types=dict(
        didx=pltpu.VMEM((NBI, CH), jnp.int32),
        ones_v=pltpu.VMEM((CH,), jnp.float32),
        zv=pltpu.VMEM((RPTD,), jnp.float32),
        acc=pltpu.VMEM_SHARED((NPD,), jnp.float32),
        isems=[pltpu.SemaphoreType.DMA for _ in range(NBI)],
        ssems=[pltpu.SemaphoreType.DMA for _ in range(NBI)],
    ),
)
def _deg_kernel(dst_hbm, out_hbm, didx, ones_v, zv, acc, isems, ssems):
  cid = lax.axis_index("c")
  sid = lax.axis_index("s")
  wid = _wid()

  _zero_vmem_1d(zv, RPTD)

  one = jnp.ones((16,), jnp.float32)
  for i in range(CH // 16):
    ones_v[pl.ds(i * 16, 16)] = one
  if CH % 16:
    ones_v[pl.ds(CH - 16, 16)] = one

  ebase = wid * EPW

  def fire_iload(j, ib):
    pltpu.async_copy(dst_hbm.at[pl.ds(ebase + j * CH, CH)], didx.at[ib],
                     isems[ib])

  def wait_iload(j, ib):
    pltpu.make_async_copy(dst_hbm.at[pl.ds(ebase + j * CH, CH)],
                          didx.at[ib], isems[ib]).wait()

  for ib in range(NBI):
    fire_iload(ib, ib)
  pltpu.sync_copy(zv, acc.at[pl.ds(sid * RPTD, RPTD)])
  plsc.subcore_barrier()

  def group(g, carry):
    for b6 in range(NBI):
      j = g * NBI + b6
      wait_iload(j, b6)
      pltpu.async_copy(ones_v, acc.at[didx.at[b6]], ssems[b6], add=True)
      pltpu.make_async_copy(ones_v, acc.at[didx.at[b6]], ssems[b6]).wait()

      @pl.when(j + NBI < NSTEP)
      def _():
        fire_iload(j + NBI, b6)

    return carry

  lax.fori_loop(0, NGRP, group, 0)
  plsc.subcore_barrier()

  pltpu.sync_copy(acc.at[pl.ds(sid * RPTD, RPTD)],
                  out_hbm.at[cid, pl.ds(sid * RPTD, RPTD)])


@functools.partial(
    pl.kernel,
    out_type=jax.ShapeDtypeStruct((NC, NPT, D), jnp.float32),
    mesh=_mesh,
    scratch_types=dict(
        sidx=pltpu.VMEM((NBI, CH), jnp.int32),
        didx=pltpu.VMEM((NBI, CH), jnp.int32),
        rows=pltpu.VMEM((NB, CH, D), jnp.float32),
        acc=pltpu.VMEM_SHARED((NP, D), jnp.float32),
        isems=[pltpu.SemaphoreType.DMA for _ in range(NBI)],
        gsems=[pltpu.SemaphoreType.DMA for _ in range(NB)],
        ssems=[pltpu.SemaphoreType.DMA for _ in range(NB)],
    ),
)
def _mp_kernel(y_hbm, src_hbm, dst_hbm, out_hbm, sidx, didx, rows, acc,
               isems, gsems, ssems):
  cid = lax.axis_index("c")
  sid = lax.axis_index("s")
  ngrp = jnp.where(cid == 0, GA, GB)
  nstep = ngrp * NBI
  ebase = jnp.where(cid == 0, sid * EPWA, NS * EPWA + sid * EPWB)

  def fire_iload(j, ib):
    pltpu.async_copy(src_hbm.at[pl.ds(ebase + j * CH, CH)], sidx.at[ib],
                     isems[ib])
    pltpu.async_copy(dst_hbm.at[pl.ds(ebase + j * CH, CH)], didx.at[ib],
                     isems[ib])

  def wait_iload(j, ib):
    pltpu.make_async_copy(src_hbm.at[pl.ds(ebase + j * CH, CH)],
                          sidx.at[ib], isems[ib]).wait()
    pltpu.make_async_copy(dst_hbm.at[pl.ds(ebase + j * CH, CH)],
                          didx.at[ib], isems[ib]).wait()

  def fire_gather(j, ib, b):
    pltpu.async_copy(y_hbm.at[sidx.at[ib]], rows.at[b], gsems[b])

  for ib in range(NBI):
    fire_iload(ib, ib)

  _zero_vmem_rows(rows.at[0], CH)
  for k in range(RPA // CH):
    pltpu.sync_copy(rows.at[0], acc.at[pl.ds(sid * RPA + k * CH, CH)])
  if RPA % CH:
    pltpu.sync_copy(rows.at[0].at[pl.ds(0, RPA % CH)],
                    acc.at[pl.ds(sid * RPA + (RPA // CH) * CH, RPA % CH)])

  @pl.when(sid == NS - 1)
  def _():
    pltpu.sync_copy(rows.at[0].at[pl.ds(0, TAIL)],
                    acc.at[pl.ds(NS * RPA, TAIL)])

  for b in range(NB):
    wait_iload(b, b)
    fire_gather(b, b, b)
  plsc.subcore_barrier()

  def group(g, carry):
    for b6 in range(NBI):
      j = g * NBI + b6
      b = b6 % NB
      pltpu.make_async_copy(y_hbm.at[sidx.at[b6]], rows.at[b],
                            gsems[b]).wait()
      pltpu.async_copy(rows.at[b], acc.at[didx.at[b6]], ssems[b], add=True)
      pltpu.make_async_copy(rows.at[b], acc.at[didx.at[b6]], ssems[b]).wait()

      @pl.when(j + NBI < nstep)
      def _():
        fire_iload(j + NBI, b6)

      @pl.when(j + NB < nstep)
      def _():
        ib2 = (b6 + NB) % NBI
        wait_iload(j + NB, ib2)
        fire_gather(j + NB, ib2, b)

    return carry

  lax.fori_loop(0, ngrp, group, 0)
  plsc.subcore_barrier()

  pltpu.sync_copy(acc.at[pl.ds(sid * RPA, RPA)],
                  out_hbm.at[cid, pl.ds(sid * RPA, RPA)])

  @pl.when(sid == NS - 1)
  def _():
    pltpu.sync_copy(acc.at[pl.ds(NS * RPA, TAIL)],
                    out_hbm.at[cid, pl.ds(NS * RPA, TAIL)])


_B = 1024


def _tc_matmul_body(x_ref, w_ref, o_ref):
  o_ref[...] = jnp.dot(x_ref[...], w_ref[...],
                       preferred_element_type=jnp.float32)


def _tc_matmul(x, w):
  return pl.pallas_call(
      _tc_matmul_body,
      grid=(NPT // _B,),
      in_specs=[
          pl.BlockSpec((_B, D), lambda i: (i, 0)),
          pl.BlockSpec((D, D), lambda i: (0, 0)),
      ],
      out_specs=pl.BlockSpec((_B, D), lambda i: (i, 0)),
      out_shape=jax.ShapeDtypeStruct((NPT, D), jnp.float32),
  )(x, w)


def _tc_scale_body(deg_ref, xw_ref, y_ref, dinv_ref):
  deg = deg_ref[0, :] + deg_ref[1, :] + 1.0
  dinv = lax.rsqrt(deg)
  dinv_ref[...] = dinv
  y_ref[...] = dinv[:, None] * xw_ref[...]


def _tc_scale(deg2, xw):
  return pl.pallas_call(
      _tc_scale_body,
      grid=(NPT // _B,),
      in_specs=[
          pl.BlockSpec((NC, _B), lambda i: (0, i)),
          pl.BlockSpec((_B, D), lambda i: (i, 0)),
      ],
      out_specs=[
          pl.BlockSpec((_B, D), lambda i: (i, 0)),
          pl.BlockSpec((_B,), lambda i: (i,)),
      ],
      out_shape=[
          jax.ShapeDtypeStruct((NPT, D), jnp.float32),
          jax.ShapeDtypeStruct((NPT,), jnp.float32),
      ],
  )(deg2, xw)


def _ln_relu(t, g, b, eps=1e-5):
  m = jnp.mean(t, axis=-1, keepdims=True)
  v = jnp.mean((t - m) * (t - m), axis=-1, keepdims=True)
  h = (t - m) * lax.rsqrt(v + eps) * g[None, :] + b[None, :]
  return jnp.maximum(h, 0.0)


def _tc_mid_body(s_ref, y_ref, dinv_ref, b_ref, g_ref, be_ref, w_ref, o_ref):
  t = s_ref[0] + s_ref[1] + y_ref[...]
  t = dinv_ref[...][:, None] * t + b_ref[...][None, :]
  h = _ln_relu(t, g_ref[...], be_ref[...])
  o_ref[...] = (dinv_ref[...][:, None]
                * jnp.dot(h, w_ref[...], preferred_element_type=jnp.float32))


def _tc_mid(s, y, dinv, b, g, be, w):
  return pl.pallas_call(
      _tc_mid_body,
      grid=(NPT // _B,),
      in_specs=[
          pl.BlockSpec((NC, _B, D), lambda i: (0, i, 0)),
          pl.BlockSpec((_B, D), lambda i: (i, 0)),
          pl.BlockSpec((_B,), lambda i: (i,)),
          pl.BlockSpec((D,), lambda i: (0,)),
          pl.BlockSpec((D,), lambda i: (0,)),
          pl.BlockSpec((D,), lambda i: (0,)),
          pl.BlockSpec((D, D), lambda i: (0, 0)),
      ],
      out_specs=pl.BlockSpec((_B, D), lambda i: (i, 0)),
      out_shape=jax.ShapeDtypeStruct((NPT, D), jnp.float32),
  )(s, y, dinv, b, g, be, w)


def _tc_final_body(s_ref, y_ref, dinv_ref, b_ref, g_ref, be_ref, o_ref):
  t = s_ref[0] + s_ref[1] + y_ref[...]
  t = dinv_ref[...][:, None] * t + b_ref[...][None, :]
  o_ref[...] = _ln_relu(t, g_ref[...], be_ref[...])


def _tc_final(s, y, dinv, b, g, be):
  return pl.pallas_call(
      _tc_final_body,
      grid=(NPT // _B,),
      in_specs=[
          pl.BlockSpec((NC, _B, D), lambda i: (0, i, 0)),
          pl.BlockSpec((_B, D), lambda i: (i, 0)),
          pl.BlockSpec((_B,), lambda i: (i,)),
          pl.BlockSpec((D,), lambda i: (0,)),
          pl.BlockSpec((D,), lambda i: (0,)),
          pl.BlockSpec((D,), lambda i: (0,)),
      ],
      out_specs=pl.BlockSpec((_B, D), lambda i: (i, 0)),
      out_shape=jax.ShapeDtypeStruct((NPT, D), jnp.float32),
  )(s, y, dinv, b, g, be)


def kernel(x, edge_index, W1, b1, g1, be1, W2, b2, g2, be2):
  pad_e = jnp.full((EP - E,), N, dtype=jnp.int32)
  src_p = jnp.concatenate([edge_index[0], pad_e])
  dst_p = jnp.concatenate([edge_index[1], pad_e])
  x_p = jnp.concatenate([x, jnp.zeros((NPT - N, D), x.dtype)], axis=0)

  deg2 = _deg_kernel(dst_p)
  xw1 = _tc_matmul(x_p, W1)
  y1, dinv = _tc_scale(deg2, xw1)
  s1 = _mp_kernel(y1, src_p, dst_p)
  y2 = _tc_mid(s1, y1, dinv, b1, g1, be1, W2)
  s2 = _mp_kernel(y2, src_p, dst_p)
  out = _tc_final(s2, y2, dinv, b2, g2, be2)
  return out[:N]


# speedup vs baseline: 20.5994x; 1.0449x over previous
import functools
import jax
import jax.numpy as jnp
from jax import lax
from jax.experimental import pallas as pl
from jax.experimental.pallas import tpu as pltpu
from jax.experimental.pallas import tpu_sc as plsc

N = 10000
D = 128
E = 320000

NC = 2
NS = 16
NW = NC * NS
CH = 120

NB = 3
NBI = 6

NP = 10016
RPA = 624
TAIL = NP - NS * RPA
NPT = 10240
NPD = NPT
RPTD = NPD // NS
EPW = ((E // NW + NBI * CH - 1) // (NBI * CH)) * (NBI * CH)
EP = EPW * NW
NSTEP = EPW // CH
NGRP = NSTEP // NBI

GA = 22
GB = 2 * NGRP - GA
EPWA = GA * NBI * CH
EPWB = GB * NBI * CH

_mesh = plsc.VectorSubcoreMesh(core_axis_name="c", subcore_axis_name="s")


def _wid():
  return lax.axis_index("s") * NC + lax.axis_index("c")


def _zero_vmem_rows(ref, nrows):
  z = jnp.zeros((16,), jnp.float32)

  def body(r, carry):
    for c in range(D // 16):
      ref[r, pl.ds(c * 16, 16)] = z
    return carry

  lax.fori_loop(0, nrows, body, 0)


def _zero_vmem_1d(ref, n):
  z = jnp.zeros((16,), jnp.float32)

  def body(i, carry):
    ref[pl.ds(i * 16, 16)] = z
    return carry

  lax.fori_loop(0, n // 16, body, 0)


@functools.partial(
    pl.kernel,
    out_type=jax.ShapeDtypeStruct((NC, NPD), jnp.float32),
    mesh=_mesh,
    scratch_types=dict(
        didx=pltpu.VMEM((NBI, CH), jnp.int32),
        ones_v=pltpu.VMEM((CH,), jnp.float32),
        zv=pltpu.VMEM((RPTD,), jnp.float32),
        acc=pltpu.VMEM_SHARED((NPD,), jnp.float32),
        isems=[pltpu.SemaphoreType.DMA for _ in range(NBI)],
        ssems=[pltpu.SemaphoreType.DMA for _ in range(NBI)],
    ),
)
def _deg_kernel(dst_hbm, out_hbm, didx, ones_v, zv, acc, isems, ssems):
  cid = lax.axis_index("c")
  sid = lax.axis_index("s")
  wid = _wid()

  _zero_vmem_1d(zv, RPTD)

  one = jnp.ones((16,), jnp.float32)
  for i in range(CH // 16):
    ones_v[pl.ds(i * 16, 16)] = one
  if CH % 16:
    ones_v[pl.ds(CH - 16, 16)] = one

  ebase = wid * EPW

  def fire_iload(j, ib):
    pltpu.async_copy(dst_hbm.at[pl.ds(ebase + j * CH, CH)], didx.at[ib],
                     isems[ib])

  def wait_iload(j, ib):
    pltpu.make_async_copy(dst_hbm.at[pl.ds(ebase + j * CH, CH)],
                          didx.at[ib], isems[ib]).wait()

  for ib in range(NBI):
    fire_iload(ib, ib)
  pltpu.sync_copy(zv, acc.at[pl.ds(sid * RPTD, RPTD)])
  plsc.subcore_barrier()

  def group(g, carry):
    for b6 in range(NBI):
      j = g * NBI + b6
      wait_iload(j, b6)
      pltpu.async_copy(ones_v, acc.at[didx.at[b6]], ssems[b6], add=True)
      pltpu.make_async_copy(ones_v, acc.at[didx.at[b6]], ssems[b6]).wait()

      @pl.when(j + NBI < NSTEP)
      def _():
        fire_iload(j + NBI, b6)

    return carry

  lax.fori_loop(0, NGRP, group, 0)
  plsc.subcore_barrier()

  pltpu.sync_copy(acc.at[pl.ds(sid * RPTD, RPTD)],
                  out_hbm.at[cid, pl.ds(sid * RPTD, RPTD)])


@functools.partial(
    pl.kernel,
    out_type=jax.ShapeDtypeStruct((NC, NPT, D), jnp.float32),
    mesh=_mesh,
    scratch_types=dict(
        sidx=pltpu.VMEM((NBI, CH), jnp.int32),
        didx=pltpu.VMEM((NBI, CH), jnp.int32),
        rows=pltpu.VMEM((NB, CH, D), jnp.float32),
        acc=pltpu.VMEM_SHARED((NP, D), jnp.float32),
        isems=[pltpu.SemaphoreType.DMA for _ in range(NBI)],
        gsems=[pltpu.SemaphoreType.DMA for _ in range(NB)],
        ssems=[pltpu.SemaphoreType.DMA for _ in range(NB)],
    ),
)
def _mp_kernel(y_hbm, src_hbm, dst_hbm, out_hbm, sidx, didx, rows, acc,
               isems, gsems, ssems):
  cid = lax.axis_index("c")
  sid = lax.axis_index("s")
  ngrp = jnp.where(cid == 0, GA, GB)
  nstep = ngrp * NBI
  ebase = jnp.where(cid == 0, sid * EPWA, NS * EPWA + sid * EPWB)

  def fire_iload(j, ib):
    pltpu.async_copy(src_hbm.at[pl.ds(ebase + j * CH, CH)], sidx.at[ib],
                     isems[ib])
    pltpu.async_copy(dst_hbm.at[pl.ds(ebase + j * CH, CH)], didx.at[ib],
                     isems[ib])

  def wait_iload(j, ib):
    pltpu.make_async_copy(src_hbm.at[pl.ds(ebase + j * CH, CH)],
                          sidx.at[ib], isems[ib]).wait()
    pltpu.make_async_copy(dst_hbm.at[pl.ds(ebase + j * CH, CH)],
                          didx.at[ib], isems[ib]).wait()

  def fire_gather(j, ib, b):
    pltpu.async_copy(y_hbm.at[sidx.at[ib]], rows.at[b], gsems[b])

  for ib in range(NBI):
    fire_iload(ib, ib)

  _zero_vmem_rows(rows.at[0], CH)
  for k in range(RPA // CH):
    pltpu.sync_copy(rows.at[0], acc.at[pl.ds(sid * RPA + k * CH, CH)])
  if RPA % CH:
    pltpu.sync_copy(rows.at[0].at[pl.ds(0, RPA % CH)],
                    acc.at[pl.ds(sid * RPA + (RPA // CH) * CH, RPA % CH)])

  @pl.when(sid == NS - 1)
  def _():
    pltpu.sync_copy(rows.at[0].at[pl.ds(0, TAIL)],
                    acc.at[pl.ds(NS * RPA, TAIL)])

  for b in range(NB):
    wait_iload(b, b)
    fire_gather(b, b, b)
  plsc.subcore_barrier()

  def group(g, carry):
    for b6 in range(NBI):
      j = g * NBI + b6
      b = b6 % NB
      pltpu.make_async_copy(y_hbm.at[sidx.at[b6]], rows.at[b],
                            gsems[b]).wait()
      pltpu.async_copy(rows.at[b], acc.at[didx.at[b6]], ssems[b], add=True)
      pltpu.make_async_copy(rows.at[b], acc.at[didx.at[b6]], ssems[b]).wait()

      @pl.when(j + NBI < nstep)
      def _():
        fire_iload(j + NBI, b6)

      @pl.when(j + NB < nstep)
      def _():
        ib2 = (b6 + NB) % NBI
        wait_iload(j + NB, ib2)
        fire_gather(j + NB, ib2, b)

    return carry

  lax.fori_loop(0, ngrp, group, 0)
  plsc.subcore_barrier()

  pltpu.sync_copy(acc.at[pl.ds(sid * RPA, RPA)],
                  out_hbm.at[cid, pl.ds(sid * RPA, RPA)])

  @pl.when(sid == NS - 1)
  def _():
    pltpu.sync_copy(acc.at[pl.ds(NS * RPA, TAIL)],
                    out_hbm.at[cid, pl.ds(NS * RPA, TAIL)])


_B = 1024


def _tc_matmul_body(x_ref, w_ref, o_ref):
  o_ref[...] = jnp.dot(x_ref[...], w_ref[...],
                       preferred_element_type=jnp.float32)


def _tc_matmul(x, w):
  return pl.pallas_call(
      _tc_matmul_body,
      grid=(NPT // _B,),
      in_specs=[
          pl.BlockSpec((_B, D), lambda i: (i, 0)),
          pl.BlockSpec((D, D), lambda i: (0, 0)),
      ],
      out_specs=pl.BlockSpec((_B, D), lambda i: (i, 0)),
      out_shape=jax.ShapeDtypeStruct((NPT, D), jnp.float32),
  )(x, w)


def _tc_scale_body(deg_ref, xw_ref, y_ref, dinv_ref):
  deg = deg_ref[0, :] + deg_ref[1, :] + 1.0
  dinv = lax.rsqrt(deg)
  dinv_ref[...] = dinv
  y_ref[...] = dinv[:, None] * xw_ref[...]


def _tc_scale(deg2, xw):
  return pl.pallas_call(
      _tc_scale_body,
      grid=(NPT // _B,),
      in_specs=[
          pl.BlockSpec((NC, _B), lambda i: (0, i)),
          pl.BlockSpec((_B, D), lambda i: (i, 0)),
      ],
      out_specs=[
          pl.BlockSpec((_B, D), lambda i: (i, 0)),
          pl.BlockSpec((_B,), lambda i: (i,)),
      ],
      out_shape=[
          jax.ShapeDtypeStruct((NPT, D), jnp.float32),
          jax.ShapeDtypeStruct((NPT,), jnp.float32),
      ],
  )(deg2, xw)


def _ln_relu(t, g, b, eps=1e-5):
  m = jnp.mean(t, axis=-1, keepdims=True)
  v = jnp.mean((t - m) * (t - m), axis=-1, keepdims=True)
  h = (t - m) * lax.rsqrt(v + eps) * g[None, :] + b[None, :]
  return jnp.maximum(h, 0.0)


def _tc_mid_body(s_ref, y_ref, dinv_ref, b_ref, g_ref, be_ref, w_ref, o_ref):
  t = s_ref[0] + s_ref[1] + y_ref[...]
  t = dinv_ref[...][:, None] * t + b_ref[...][None, :]
  h = _ln_relu(t, g_ref[...], be_ref[...])
  o_ref[...] = (dinv_ref[...][:, None]
                * jnp.dot(h, w_ref[...], preferred_element_type=jnp.float32))


def _tc_mid(s, y, dinv, b, g, be, w):
  return pl.pallas_call(
      _tc_mid_body,
      grid=(NPT // _B,),
      in_specs=[
          pl.BlockSpec((NC, _B, D), lambda i: (0, i, 0)),
          pl.BlockSpec((_B, D), lambda i: (i, 0)),
          pl.BlockSpec((_B,), lambda i: (i,)),
          pl.BlockSpec((D,), lambda i: (0,)),
          pl.BlockSpec((D,), lambda i: (0,)),
          pl.BlockSpec((D,), lambda i: (0,)),
          pl.BlockSpec((D, D), lambda i: (0, 0)),
      ],
      out_specs=pl.BlockSpec((_B, D), lambda i: (i, 0)),
      out_shape=jax.ShapeDtypeStruct((NPT, D), jnp.float32),
  )(s, y, dinv, b, g, be, w)


def _tc_final_body(s_ref, y_ref, dinv_ref, b_ref, g_ref, be_ref, o_ref):
  t = s_ref[0] + s_ref[1] + y_ref[...]
  t = dinv_ref[...][:, None] * t + b_ref[...][None, :]
  o_ref[...] = _ln_relu(t, g_ref[...], be_ref[...])


def _tc_final(s, y, dinv, b, g, be):
  return pl.pallas_call(
      _tc_final_body,
      grid=(NPT // _B,),
      in_specs=[
          pl.BlockSpec((NC, _B, D), lambda i: (0, i, 0)),
          pl.BlockSpec((_B, D), lambda i: (i, 0)),
          pl.BlockSpec((_B,), lambda i: (i,)),
          pl.BlockSpec((D,), lambda i: (0,)),
          pl.BlockSpec((D,), lambda i: (0,)),
          pl.BlockSpec((D,), lambda i: (0,)),
      ],
      out_specs=pl.BlockSpec((_B, D), lambda i: (i, 0)),
      out_shape=jax.ShapeDtypeStruct((NPT, D), jnp.float32),
  )(s, y, dinv, b, g, be)


def kernel(x, edge_index, W1, b1, g1, be1, W2, b2, g2, be2):
  pad_e = jnp.full((EP - E,), N, dtype=jnp.int32)
  src_p = jnp.concatenate([edge_index[0], pad_e])
  dst_p = jnp.concatenate([edge_index[1], pad_e])
  x_p = jnp.concatenate([x, jnp.zeros((NPT - N, D), x.dtype)], axis=0)

  deg2 = _deg_kernel(dst_p)
  xw1 = _tc_matmul(x_p, W1)
  y1, dinv = _tc_scale(deg2, xw1)
  s1 = _mp_kernel(y1, src_p, dst_p)
  y2 = _tc_mid(s1, y1, dinv, b1, g1, be1, W2)
  s2 = _mp_kernel(y2, src_p, dst_p)
  out = _tc_final(s2, y2, dinv, b2, g2, be2)
  return out[:N]


# speedup vs baseline: 21.6434x; 1.0507x over previous
import functools
import jax
import jax.numpy as jnp
from jax import lax
from jax.experimental import pallas as pl
from jax.experimental.pallas import tpu as pltpu
from jax.experimental.pallas import tpu_sc as plsc

N = 10000
D = 128
E = 320000

NC = 2
NS = 16
NW = NC * NS
CH = 120

NB = 3
NBI = 6

NP = 10016
RPA = 624
TAIL = NP - NS * RPA
NPT = 10240
NPD = NPT
RPTD = NPD // NS
EPW = ((E // NW + NBI * CH - 1) // (NBI * CH)) * (NBI * CH)
EP = EPW * NW
NSTEP = EPW // CH
NGRP = NSTEP // NBI

GA = 24
GB = 2 * NGRP - GA
EPWA = GA * NBI * CH
EPWB = GB * NBI * CH

_mesh = plsc.VectorSubcoreMesh(core_axis_name="c", subcore_axis_name="s")


def _wid():
  return lax.axis_index("s") * NC + lax.axis_index("c")


def _zero_vmem_rows(ref, nrows):
  z = jnp.zeros((16,), jnp.float32)

  def body(r, carry):
    for c in range(D // 16):
      ref[r, pl.ds(c * 16, 16)] = z
    return carry

  lax.fori_loop(0, nrows, body, 0)


def _zero_vmem_1d(ref, n):
  z = jnp.zeros((16,), jnp.float32)

  def body(i, carry):
    ref[pl.ds(i * 16, 16)] = z
    return carry

  lax.fori_loop(0, n // 16, body, 0)


@functools.partial(
    pl.kernel,
    out_type=jax.ShapeDtypeStruct((NC, NPD), jnp.float32),
    mesh=_mesh,
    scratch_types=dict(
        didx=pltpu.VMEM((NBI, CH), jnp.int32),
        ones_v=pltpu.VMEM((CH,), jnp.float32),
        zv=pltpu.VMEM((RPTD,), jnp.float32),
        acc=pltpu.VMEM_SHARED((NPD,), jnp.float32),
        isems=[pltpu.SemaphoreType.DMA for _ in range(NBI)],
        ssems=[pltpu.SemaphoreType.DMA for _ in range(NBI)],
    ),
)
def _deg_kernel(dst_hbm, out_hbm, didx, ones_v, zv, acc, isems, ssems):
  cid = lax.axis_index("c")
  sid = lax.axis_index("s")
  wid = _wid()

  _zero_vmem_1d(zv, RPTD)

  one = jnp.ones((16,), jnp.float32)
  for i in range(CH // 16):
    ones_v[pl.ds(i * 16, 16)] = one
  if CH % 16:
    ones_v[pl.ds(CH - 16, 16)] = one

  ebase = wid * EPW

  def fire_iload(j, ib):
    pltpu.async_copy(dst_hbm.at[pl.ds(ebase + j * CH, CH)], didx.at[ib],
                     isems[ib])

  def wait_iload(j, ib):
    pltpu.make_async_copy(dst_hbm.at[pl.ds(ebase + j * CH, CH)],
                          didx.at[ib], isems[ib]).wait()

  for ib in range(NBI):
    fire_iload(ib, ib)
  pltpu.sync_copy(zv, acc.at[pl.ds(sid * RPTD, RPTD)])
  plsc.subcore_barrier()

  def group(g, carry):
    for b6 in range(NBI):
      j = g * NBI + b6
      wait_iload(j, b6)
      pltpu.async_copy(ones_v, acc.at[didx.at[b6]], ssems[b6], add=True)
      pltpu.make_async_copy(ones_v, acc.at[didx.at[b6]], ssems[b6]).wait()

      @pl.when(j + NBI < NSTEP)
      def _():
        fire_iload(j + NBI, b6)

    return carry

  lax.fori_loop(0, NGRP, group, 0)
  plsc.subcore_barrier()

  pltpu.sync_copy(acc.at[pl.ds(sid * RPTD, RPTD)],
                  out_hbm.at[cid, pl.ds(sid * RPTD, RPTD)])


@functools.partial(
    pl.kernel,
    out_type=jax.ShapeDtypeStruct((NC, NPT, D), jnp.float32),
    mesh=_mesh,
    scratch_types=dict(
        sidx=pltpu.VMEM((NBI, CH), jnp.int32),
        didx=pltpu.VMEM((NBI, CH), jnp.int32),
        rows=pltpu.VMEM((NB, CH, D), jnp.float32),
        acc=pltpu.VMEM_SHARED((NP, D), jnp.float32),
        isems=[pltpu.SemaphoreType.DMA for _ in range(NBI)],
        gsems=[pltpu.SemaphoreType.DMA for _ in range(NB)],
        ssems=[pltpu.SemaphoreType.DMA for _ in range(NB)],
    ),
)
def _mp_kernel(y_hbm, src_hbm, dst_hbm, out_hbm, sidx, didx, rows, acc,
               isems, gsems, ssems):
  cid = lax.axis_index("c")
  sid = lax.axis_index("s")
  ngrp = jnp.where(cid == 0, GA, GB)
  nstep = ngrp * NBI
  ebase = jnp.where(cid == 0, sid * EPWA, NS * EPWA + sid * EPWB)

  def fire_iload(j, ib):
    pltpu.async_copy(src_hbm.at[pl.ds(ebase + j * CH, CH)], sidx.at[ib],
                     isems[ib])
    pltpu.async_copy(dst_hbm.at[pl.ds(ebase + j * CH, CH)], didx.at[ib],
                     isems[ib])

  def wait_iload(j, ib):
    pltpu.make_async_copy(src_hbm.at[pl.ds(ebase + j * CH, CH)],
                          sidx.at[ib], isems[ib]).wait()
    pltpu.make_async_copy(dst_hbm.at[pl.ds(ebase + j * CH, CH)],
                          didx.at[ib], isems[ib]).wait()

  def fire_gather(j, ib, b):
    pltpu.async_copy(y_hbm.at[sidx.at[ib]], rows.at[b], gsems[b])

  for ib in range(NBI):
    fire_iload(ib, ib)

  _zero_vmem_rows(rows.at[0], CH)
  for k in range(RPA // CH):
    pltpu.sync_copy(rows.at[0], acc.at[pl.ds(sid * RPA + k * CH, CH)])
  if RPA % CH:
    pltpu.sync_copy(rows.at[0].at[pl.ds(0, RPA % CH)],
                    acc.at[pl.ds(sid * RPA + (RPA // CH) * CH, RPA % CH)])

  @pl.when(sid == NS - 1)
  def _():
    pltpu.sync_copy(rows.at[0].at[pl.ds(0, TAIL)],
                    acc.at[pl.ds(NS * RPA, TAIL)])

  for b in range(NB):
    wait_iload(b, b)
    fire_gather(b, b, b)
  plsc.subcore_barrier()

  def group(g, carry):
    for b6 in range(NBI):
      j = g * NBI + b6
      b = b6 % NB
      pltpu.make_async_copy(y_hbm.at[sidx.at[b6]], rows.at[b],
                            gsems[b]).wait()
      pltpu.async_copy(rows.at[b], acc.at[didx.at[b6]], ssems[b], add=True)
      pltpu.make_async_copy(rows.at[b], acc.at[didx.at[b6]], ssems[b]).wait()

      @pl.when(j + NBI < nstep)
      def _():
        fire_iload(j + NBI, b6)

      @pl.when(j + NB < nstep)
      def _():
        ib2 = (b6 + NB) % NBI
        wait_iload(j + NB, ib2)
        fire_gather(j + NB, ib2, b)

    return carry

  lax.fori_loop(0, ngrp, group, 0)
  plsc.subcore_barrier()

  pltpu.sync_copy(acc.at[pl.ds(sid * RPA, RPA)],
                  out_hbm.at[cid, pl.ds(sid * RPA, RPA)])

  @pl.when(sid == NS - 1)
  def _():
    pltpu.sync_copy(acc.at[pl.ds(NS * RPA, TAIL)],
                    out_hbm.at[cid, pl.ds(NS * RPA, TAIL)])


_B = 1024


def _tc_matmul_body(x_ref, w_ref, o_ref):
  o_ref[...] = jnp.dot(x_ref[...], w_ref[...],
                       preferred_element_type=jnp.float32)


def _tc_matmul(x, w):
  return pl.pallas_call(
      _tc_matmul_body,
      grid=(NPT // _B,),
      in_specs=[
          pl.BlockSpec((_B, D), lambda i: (i, 0)),
          pl.BlockSpec((D, D), lambda i: (0, 0)),
      ],
      out_specs=pl.BlockSpec((_B, D), lambda i: (i, 0)),
      out_shape=jax.ShapeDtypeStruct((NPT, D), jnp.float32),
  )(x, w)


def _tc_scale_body(deg_ref, xw_ref, y_ref, dinv_ref):
  deg = deg_ref[0, :] + deg_ref[1, :] + 1.0
  dinv = lax.rsqrt(deg)
  dinv_ref[...] = dinv
  y_ref[...] = dinv[:, None] * xw_ref[...]


def _tc_scale(deg2, xw):
  return pl.pallas_call(
      _tc_scale_body,
      grid=(NPT // _B,),
      in_specs=[
          pl.BlockSpec((NC, _B), lambda i: (0, i)),
          pl.BlockSpec((_B, D), lambda i: (i, 0)),
      ],
      out_specs=[
          pl.BlockSpec((_B, D), lambda i: (i, 0)),
          pl.BlockSpec((_B,), lambda i: (i,)),
      ],
      out_shape=[
          jax.ShapeDtypeStruct((NPT, D), jnp.float32),
          jax.ShapeDtypeStruct((NPT,), jnp.float32),
      ],
  )(deg2, xw)


def _ln_relu(t, g, b, eps=1e-5):
  m = jnp.mean(t, axis=-1, keepdims=True)
  v = jnp.mean((t - m) * (t - m), axis=-1, keepdims=True)
  h = (t - m) * lax.rsqrt(v + eps) * g[None, :] + b[None, :]
  return jnp.maximum(h, 0.0)


def _tc_mid_body(s_ref, y_ref, dinv_ref, b_ref, g_ref, be_ref, w_ref, o_ref):
  t = s_ref[0] + s_ref[1] + y_ref[...]
  t = dinv_ref[...][:, None] * t + b_ref[...][None, :]
  h = _ln_relu(t, g_ref[...], be_ref[...])
  o_ref[...] = (dinv_ref[...][:, None]
                * jnp.dot(h, w_ref[...], preferred_element_type=jnp.float32))


def _tc_mid(s, y, dinv, b, g, be, w):
  return pl.pallas_call(
      _tc_mid_body,
      grid=(NPT // _B,),
      in_specs=[
          pl.BlockSpec((NC, _B, D), lambda i: (0, i, 0)),
          pl.BlockSpec((_B, D), lambda i: (i, 0)),
          pl.BlockSpec((_B,), lambda i: (i,)),
          pl.BlockSpec((D,), lambda i: (0,)),
          pl.BlockSpec((D,), lambda i: (0,)),
          pl.BlockSpec((D,), lambda i: (0,)),
          pl.BlockSpec((D, D), lambda i: (0, 0)),
      ],
      out_specs=pl.BlockSpec((_B, D), lambda i: (i, 0)),
      out_shape=jax.ShapeDtypeStruct((NPT, D), jnp.float32),
  )(s, y, dinv, b, g, be, w)


def _tc_final_body(s_ref, y_ref, dinv_ref, b_ref, g_ref, be_ref, o_ref):
  t = s_ref[0] + s_ref[1] + y_ref[...]
  t = dinv_ref[...][:, None] * t + b_ref[...][None, :]
  o_ref[...] = _ln_relu(t, g_ref[...], be_ref[...])


def _tc_final(s, y, dinv, b, g, be):
  return pl.pallas_call(
      _tc_final_body,
      grid=(NPT // _B,),
      in_specs=[
          pl.BlockSpec((NC, _B, D), lambda i: (0, i, 0)),
          pl.BlockSpec((_B, D), lambda i: (i, 0)),
          pl.BlockSpec((_B,), lambda i: (i,)),
          pl.BlockSpec((D,), lambda i: (0,)),
          pl.BlockSpec((D,), lambda i: (0,)),
          pl.BlockSpec((D,), lambda i: (0,)),
      ],
      out_specs=pl.BlockSpec((_B, D), lambda i: (i, 0)),
      out_shape=jax.ShapeDtypeStruct((NPT, D), jnp.float32),
  )(s, y, dinv, b, g, be)


def kernel(x, edge_index, W1, b1, g1, be1, W2, b2, g2, be2):
  pad_e = jnp.full((EP - E,), N, dtype=jnp.int32)
  src_p = jnp.concatenate([edge_index[0], pad_e])
  dst_p = jnp.concatenate([edge_index[1], pad_e])
  x_p = jnp.concatenate([x, jnp.zeros((NPT - N, D), x.dtype)], axis=0)

  deg2 = _deg_kernel(dst_p)
  xw1 = _tc_matmul(x_p, W1)
  y1, dinv = _tc_scale(deg2, xw1)
  s1 = _mp_kernel(y1, src_p, dst_p)
  y2 = _tc_mid(s1, y1, dinv, b1, g1, be1, W2)
  s2 = _mp_kernel(y2, src_p, dst_p)
  out = _tc_final(s2, y2, dinv, b2, g2, be2)
  return out[:N]


# speedup vs baseline: 22.2883x; 1.0298x over previous
import functools
import jax
import jax.numpy as jnp
from jax import lax
from jax.experimental import pallas as pl
from jax.experimental.pallas import tpu as pltpu
from jax.experimental.pallas import tpu_sc as plsc

N = 10000
D = 128
E = 320000

NC = 2
NS = 16
NW = NC * NS
CH = 120

NB = 3
NBI = 6

NP = 10016
RPA = 624
TAIL = NP - NS * RPA
NPT = 10240
NPD = NPT
RPTD = NPD // NS
EPW = ((E // NW + NBI * CH - 1) // (NBI * CH)) * (NBI * CH)
EP = EPW * NW
NSTEP = EPW // CH
NGRP = NSTEP // NBI

GA = 26
GB = 2 * NGRP - GA
EPWA = GA * NBI * CH
EPWB = GB * NBI * CH

_mesh = plsc.VectorSubcoreMesh(core_axis_name="c", subcore_axis_name="s")


def _wid():
  return lax.axis_index("s") * NC + lax.axis_index("c")


def _zero_vmem_rows(ref, nrows):
  z = jnp.zeros((16,), jnp.float32)

  def body(r, carry):
    for c in range(D // 16):
      ref[r, pl.ds(c * 16, 16)] = z
    return carry

  lax.fori_loop(0, nrows, body, 0)


def _zero_vmem_1d(ref, n):
  z = jnp.zeros((16,), jnp.float32)

  def body(i, carry):
    ref[pl.ds(i * 16, 16)] = z
    return carry

  lax.fori_loop(0, n // 16, body, 0)


@functools.partial(
    pl.kernel,
    out_type=jax.ShapeDtypeStruct((NC, NPD), jnp.float32),
    mesh=_mesh,
    scratch_types=dict(
        didx=pltpu.VMEM((NBI, CH), jnp.int32),
        ones_v=pltpu.VMEM((CH,), jnp.float32),
        zv=pltpu.VMEM((RPTD,), jnp.float32),
        acc=pltpu.VMEM_SHARED((NPD,), jnp.float32),
        isems=[pltpu.SemaphoreType.DMA for _ in range(NBI)],
        ssems=[pltpu.SemaphoreType.DMA for _ in range(NBI)],
    ),
)
def _deg_kernel(dst_hbm, out_hbm, didx, ones_v, zv, acc, isems, ssems):
  cid = lax.axis_index("c")
  sid = lax.axis_index("s")
  wid = _wid()

  _zero_vmem_1d(zv, RPTD)

  one = jnp.ones((16,), jnp.float32)
  for i in range(CH // 16):
    ones_v[pl.ds(i * 16, 16)] = one
  if CH % 16:
    ones_v[pl.ds(CH - 16, 16)] = one

  ebase = wid * EPW

  def fire_iload(j, ib):
    pltpu.async_copy(dst_hbm.at[pl.ds(ebase + j * CH, CH)], didx.at[ib],
                     isems[ib])

  def wait_iload(j, ib):
    pltpu.make_async_copy(dst_hbm.at[pl.ds(ebase + j * CH, CH)],
                          didx.at[ib], isems[ib]).wait()

  for ib in range(NBI):
    fire_iload(ib, ib)
  pltpu.sync_copy(zv, acc.at[pl.ds(sid * RPTD, RPTD)])
  plsc.subcore_barrier()

  def group(g, carry):
    for b6 in range(NBI):
      j = g * NBI + b6
      wait_iload(j, b6)
      pltpu.async_copy(ones_v, acc.at[didx.at[b6]], ssems[b6], add=True)
      pltpu.make_async_copy(ones_v, acc.at[didx.at[b6]], ssems[b6]).wait()

      @pl.when(j + NBI < NSTEP)
      def _():
        fire_iload(j + NBI, b6)

    return carry

  lax.fori_loop(0, NGRP, group, 0)
  plsc.subcore_barrier()

  pltpu.sync_copy(acc.at[pl.ds(sid * RPTD, RPTD)],
                  out_hbm.at[cid, pl.ds(sid * RPTD, RPTD)])


@functools.partial(
    pl.kernel,
    out_type=jax.ShapeDtypeStruct((NC, NPT, D), jnp.float32),
    mesh=_mesh,
    scratch_types=dict(
        sidx=pltpu.VMEM((NBI, CH), jnp.int32),
        didx=pltpu.VMEM((NBI, CH), jnp.int32),
        rows=pltpu.VMEM((NB, CH, D), jnp.float32),
        acc=pltpu.VMEM_SHARED((NP, D), jnp.float32),
        isems=[pltpu.SemaphoreType.DMA for _ in range(NBI)],
        gsems=[pltpu.SemaphoreType.DMA for _ in range(NB)],
        ssems=[pltpu.SemaphoreType.DMA for _ in range(NB)],
    ),
)
def _mp_kernel(y_hbm, src_hbm, dst_hbm, out_hbm, sidx, didx, rows, acc,
               isems, gsems, ssems):
  cid = lax.axis_index("c")
  sid = lax.axis_index("s")
  ngrp = jnp.where(cid == 0, GA, GB)
  nstep = ngrp * NBI
  ebase = jnp.where(cid == 0, sid * EPWA, NS * EPWA + sid * EPWB)

  def fire_iload(j, ib):
    pltpu.async_copy(src_hbm.at[pl.ds(ebase + j * CH, CH)], sidx.at[ib],
                     isems[ib])
    pltpu.async_copy(dst_hbm.at[pl.ds(ebase + j * CH, CH)], didx.at[ib],
                     isems[ib])

  def wait_iload(j, ib):
    pltpu.make_async_copy(src_hbm.at[pl.ds(ebase + j * CH, CH)],
                          sidx.at[ib], isems[ib]).wait()
    pltpu.make_async_copy(dst_hbm.at[pl.ds(ebase + j * CH, CH)],
                          didx.at[ib], isems[ib]).wait()

  def fire_gather(j, ib, b):
    pltpu.async_copy(y_hbm.at[sidx.at[ib]], rows.at[b], gsems[b])

  for ib in range(NBI):
    fire_iload(ib, ib)

  _zero_vmem_rows(rows.at[0], CH)
  for k in range(RPA // CH):
    pltpu.sync_copy(rows.at[0], acc.at[pl.ds(sid * RPA + k * CH, CH)])
  if RPA % CH:
    pltpu.sync_copy(rows.at[0].at[pl.ds(0, RPA % CH)],
                    acc.at[pl.ds(sid * RPA + (RPA // CH) * CH, RPA % CH)])

  @pl.when(sid == NS - 1)
  def _():
    pltpu.sync_copy(rows.at[0].at[pl.ds(0, TAIL)],
                    acc.at[pl.ds(NS * RPA, TAIL)])

  for b in range(NB):
    wait_iload(b, b)
    fire_gather(b, b, b)
  plsc.subcore_barrier()

  def group(g, carry):
    for b6 in range(NBI):
      j = g * NBI + b6
      b = b6 % NB
      pltpu.make_async_copy(y_hbm.at[sidx.at[b6]], rows.at[b],
                            gsems[b]).wait()
      pltpu.async_copy(rows.at[b], acc.at[didx.at[b6]], ssems[b], add=True)
      pltpu.make_async_copy(rows.at[b], acc.at[didx.at[b6]], ssems[b]).wait()

      @pl.when(j + NBI < nstep)
      def _():
        fire_iload(j + NBI, b6)

      @pl.when(j + NB < nstep)
      def _():
        ib2 = (b6 + NB) % NBI
        wait_iload(j + NB, ib2)
        fire_gather(j + NB, ib2, b)

    return carry

  lax.fori_loop(0, ngrp, group, 0)
  plsc.subcore_barrier()

  pltpu.sync_copy(acc.at[pl.ds(sid * RPA, RPA)],
                  out_hbm.at[cid, pl.ds(sid * RPA, RPA)])

  @pl.when(sid == NS - 1)
  def _():
    pltpu.sync_copy(acc.at[pl.ds(NS * RPA, TAIL)],
                    out_hbm.at[cid, pl.ds(NS * RPA, TAIL)])


_B = 1024


def _tc_matmul_body(x_ref, w_ref, o_ref):
  o_ref[...] = jnp.dot(x_ref[...], w_ref[...],
                       preferred_element_type=jnp.float32)


def _tc_matmul(x, w):
  return pl.pallas_call(
      _tc_matmul_body,
      grid=(NPT // _B,),
      in_specs=[
          pl.BlockSpec((_B, D), lambda i: (i, 0)),
          pl.BlockSpec((D, D), lambda i: (0, 0)),
      ],
      out_specs=pl.BlockSpec((_B, D), lambda i: (i, 0)),
      out_shape=jax.ShapeDtypeStruct((NPT, D), jnp.float32),
  )(x, w)


def _tc_scale_body(deg_ref, xw_ref, y_ref, dinv_ref):
  deg = deg_ref[0, :] + deg_ref[1, :] + 1.0
  dinv = lax.rsqrt(deg)
  dinv_ref[...] = dinv
  y_ref[...] = dinv[:, None] * xw_ref[...]


def _tc_scale(deg2, xw):
  return pl.pallas_call(
      _tc_scale_body,
      grid=(NPT // _B,),
      in_specs=[
          pl.BlockSpec((NC, _B), lambda i: (0, i)),
          pl.BlockSpec((_B, D), lambda i: (i, 0)),
      ],
      out_specs=[
          pl.BlockSpec((_B, D), lambda i: (i, 0)),
          pl.BlockSpec((_B,), lambda i: (i,)),
      ],
      out_shape=[
          jax.ShapeDtypeStruct((NPT, D), jnp.float32),
          jax.ShapeDtypeStruct((NPT,), jnp.float32),
      ],
  )(deg2, xw)


def _ln_relu(t, g, b, eps=1e-5):
  m = jnp.mean(t, axis=-1, keepdims=True)
  v = jnp.mean((t - m) * (t - m), axis=-1, keepdims=True)
  h = (t - m) * lax.rsqrt(v + eps) * g[None, :] + b[None, :]
  return jnp.maximum(h, 0.0)


def _tc_mid_body(s_ref, y_ref, dinv_ref, b_ref, g_ref, be_ref, w_ref, o_ref):
  t = s_ref[0] + s_ref[1] + y_ref[...]
  t = dinv_ref[...][:, None] * t + b_ref[...][None, :]
  h = _ln_relu(t, g_ref[...], be_ref[...])
  o_ref[...] = (dinv_ref[...][:, None]
                * jnp.dot(h, w_ref[...], preferred_element_type=jnp.float32))


def _tc_mid(s, y, dinv, b, g, be, w):
  return pl.pallas_call(
      _tc_mid_body,
      grid=(NPT // _B,),
      in_specs=[
          pl.BlockSpec((NC, _B, D), lambda i: (0, i, 0)),
          pl.BlockSpec((_B, D), lambda i: (i, 0)),
          pl.BlockSpec((_B,), lambda i: (i,)),
          pl.BlockSpec((D,), lambda i: (0,)),
          pl.BlockSpec((D,), lambda i: (0,)),
          pl.BlockSpec((D,), lambda i: (0,)),
          pl.BlockSpec((D, D), lambda i: (0, 0)),
      ],
      out_specs=pl.BlockSpec((_B, D), lambda i: (i, 0)),
      out_shape=jax.ShapeDtypeStruct((NPT, D), jnp.float32),
  )(s, y, dinv, b, g, be, w)


def _tc_final_body(s_ref, y_ref, dinv_ref, b_ref, g_ref, be_ref, o_ref):
  t = s_ref[0] + s_ref[1] + y_ref[...]
  t = dinv_ref[...][:, None] * t + b_ref[...][None, :]
  o_ref[...] = _ln_relu(t, g_ref[...], be_ref[...])


def _tc_final(s, y, dinv, b, g, be):
  return pl.pallas_call(
      _tc_final_body,
      grid=(NPT // _B,),
      in_specs=[
          pl.BlockSpec((NC, _B, D), lambda i: (0, i, 0)),
          pl.BlockSpec((_B, D), lambda i: (i, 0)),
          pl.BlockSpec((_B,), lambda i: (i,)),
          pl.BlockSpec((D,), lambda i: (0,)),
          pl.BlockSpec((D,), lambda i: (0,)),
          pl.BlockSpec((D,), lambda i: (0,)),
      ],
      out_specs=pl.BlockSpec((_B, D), lambda i: (i, 0)),
      out_shape=jax.ShapeDtypeStruct((NPT, D), jnp.float32),
  )(s, y, dinv, b, g, be)


def kernel(x, edge_index, W1, b1, g1, be1, W2, b2, g2, be2):
  pad_e = jnp.full((EP - E,), N, dtype=jnp.int32)
  src_p = jnp.concatenate([edge_index[0], pad_e])
  dst_p = jnp.concatenate([edge_index[1], pad_e])
  x_p = jnp.concatenate([x, jnp.zeros((NPT - N, D), x.dtype)], axis=0)

  deg2 = _deg_kernel(dst_p)
  xw1 = _tc_matmul(x_p, W1)
  y1, dinv = _tc_scale(deg2, xw1)
  s1 = _mp_kernel(y1, src_p, dst_p)
  y2 = _tc_mid(s1, y1, dinv, b1, g1, be1, W2)
  s2 = _mp_kernel(y2, src_p, dst_p)
  out = _tc_final(s2, y2, dinv, b2, g2, be2)
  return out[:N]


# speedup vs baseline: 22.5234x; 1.0105x over previous
import functools
import jax
import jax.numpy as jnp
from jax import lax
from jax.experimental import pallas as pl
from jax.experimental.pallas import tpu as pltpu
from jax.experimental.pallas import tpu_sc as plsc

N = 10000
D = 128
E = 320000

NC = 2
NS = 16
NW = NC * NS
CH = 120

NB = 3
NBI = 6

NP = 10016
RPA = 624
TAIL = NP - NS * RPA
NPT = 10240
NPD = NPT
RPTD = NPD // NS
EPW = ((E // NW + NBI * CH - 1) // (NBI * CH)) * (NBI * CH)
EP = EPW * NW
NSTEP = EPW // CH
NGRP = NSTEP // NBI

GA = 27
GB = 2 * NGRP - GA
EPWA = GA * NBI * CH
EPWB = GB * NBI * CH

_mesh = plsc.VectorSubcoreMesh(core_axis_name="c", subcore_axis_name="s")


def _wid():
  return lax.axis_index("s") * NC + lax.axis_index("c")


def _zero_vmem_rows(ref, nrows):
  z = jnp.zeros((16,), jnp.float32)

  def body(r, carry):
    for c in range(D // 16):
      ref[r, pl.ds(c * 16, 16)] = z
    return carry

  lax.fori_loop(0, nrows, body, 0)


def _zero_vmem_1d(ref, n):
  z = jnp.zeros((16,), jnp.float32)

  def body(i, carry):
    ref[pl.ds(i * 16, 16)] = z
    return carry

  lax.fori_loop(0, n // 16, body, 0)


@functools.partial(
    pl.kernel,
    out_type=jax.ShapeDtypeStruct((NC, NPD), jnp.float32),
    mesh=_mesh,
    scratch_types=dict(
        didx=pltpu.VMEM((NBI, CH), jnp.int32),
        ones_v=pltpu.VMEM((CH,), jnp.float32),
        zv=pltpu.VMEM((RPTD,), jnp.float32),
        acc=pltpu.VMEM_SHARED((NPD,), jnp.float32),
        isems=[pltpu.SemaphoreType.DMA for _ in range(NBI)],
        ssems=[pltpu.SemaphoreType.DMA for _ in range(NBI)],
    ),
)
def _deg_kernel(dst_hbm, out_hbm, didx, ones_v, zv, acc, isems, ssems):
  cid = lax.axis_index("c")
  sid = lax.axis_index("s")
  wid = _wid()

  _zero_vmem_1d(zv, RPTD)

  one = jnp.ones((16,), jnp.float32)
  for i in range(CH // 16):
    ones_v[pl.ds(i * 16, 16)] = one
  if CH % 16:
    ones_v[pl.ds(CH - 16, 16)] = one

  ebase = wid * EPW

  def fire_iload(j, ib):
    pltpu.async_copy(dst_hbm.at[pl.ds(ebase + j * CH, CH)], didx.at[ib],
                     isems[ib])

  def wait_iload(j, ib):
    pltpu.make_async_copy(dst_hbm.at[pl.ds(ebase + j * CH, CH)],
                          didx.at[ib], isems[ib]).wait()

  for ib in range(NBI):
    fire_iload(ib, ib)
  pltpu.sync_copy(zv, acc.at[pl.ds(sid * RPTD, RPTD)])
  plsc.subcore_barrier()

  def group(g, carry):
    for b6 in range(NBI):
      j = g * NBI + b6
      wait_iload(j, b6)
      pltpu.async_copy(ones_v, acc.at[didx.at[b6]], ssems[b6], add=True)
      pltpu.make_async_copy(ones_v, acc.at[didx.at[b6]], ssems[b6]).wait()

      @pl.when(j + NBI < NSTEP)
      def _():
        fire_iload(j + NBI, b6)

    return carry

  lax.fori_loop(0, NGRP, group, 0)
  plsc.subcore_barrier()

  pltpu.sync_copy(acc.at[pl.ds(sid * RPTD, RPTD)],
                  out_hbm.at[cid, pl.ds(sid * RPTD, RPTD)])


@functools.partial(
    pl.kernel,
    out_type=jax.ShapeDtypeStruct((NC, NPT, D), jnp.float32),
    mesh=_mesh,
    scratch_types=dict(
        sidx=pltpu.VMEM((NBI, CH), jnp.int32),
        didx=pltpu.VMEM((NBI, CH), jnp.int32),
        rows=pltpu.VMEM((NB, CH, D), jnp.float32),
        acc=pltpu.VMEM_SHARED((NP, D), jnp.float32),
        isems=[pltpu.SemaphoreType.DMA for _ in range(NBI)],
        gsems=[pltpu.SemaphoreType.DMA for _ in range(NB)],
        ssems=[pltpu.SemaphoreType.DMA for _ in range(NB)],
    ),
)
def _mp_kernel(y_hbm, src_hbm, dst_hbm, out_hbm, sidx, didx, rows, acc,
               isems, gsems, ssems):
  cid = lax.axis_index("c")
  sid = lax.axis_index("s")
  ngrp = jnp.where(cid == 0, GA, GB)
  nstep = ngrp * NBI
  ebase = jnp.where(cid == 0, sid * EPWA, NS * EPWA + sid * EPWB)

  def fire_iload(j, ib):
    pltpu.async_copy(src_hbm.at[pl.ds(ebase + j * CH, CH)], sidx.at[ib],
                     isems[ib])
    pltpu.async_copy(dst_hbm.at[pl.ds(ebase + j * CH, CH)], didx.at[ib],
                     isems[ib])

  def wait_iload(j, ib):
    pltpu.make_async_copy(src_hbm.at[pl.ds(ebase + j * CH, CH)],
                          sidx.at[ib], isems[ib]).wait()
    pltpu.make_async_copy(dst_hbm.at[pl.ds(ebase + j * CH, CH)],
                          didx.at[ib], isems[ib]).wait()

  def fire_gather(j, ib, b):
    pltpu.async_copy(y_hbm.at[sidx.at[ib]], rows.at[b], gsems[b])

  for ib in range(NBI):
    fire_iload(ib, ib)

  _zero_vmem_rows(rows.at[0], CH)
  for k in range(RPA // CH):
    pltpu.sync_copy(rows.at[0], acc.at[pl.ds(sid * RPA + k * CH, CH)])
  if RPA % CH:
    pltpu.sync_copy(rows.at[0].at[pl.ds(0, RPA % CH)],
                    acc.at[pl.ds(sid * RPA + (RPA // CH) * CH, RPA % CH)])

  @pl.when(sid == NS - 1)
  def _():
    pltpu.sync_copy(rows.at[0].at[pl.ds(0, TAIL)],
                    acc.at[pl.ds(NS * RPA, TAIL)])

  for b in range(NB):
    wait_iload(b, b)
    fire_gather(b, b, b)
  plsc.subcore_barrier()

  def group(g, carry):
    for b6 in range(NBI):
      j = g * NBI + b6
      b = b6 % NB
      pltpu.make_async_copy(y_hbm.at[sidx.at[b6]], rows.at[b],
                            gsems[b]).wait()
      pltpu.async_copy(rows.at[b], acc.at[didx.at[b6]], ssems[b], add=True)
      pltpu.make_async_copy(rows.at[b], acc.at[didx.at[b6]], ssems[b]).wait()

      @pl.when(j + NBI < nstep)
      def _():
        fire_iload(j + NBI, b6)

      @pl.when(j + NB < nstep)
      def _():
        ib2 = (b6 + NB) % NBI
        wait_iload(j + NB, ib2)
        fire_gather(j + NB, ib2, b)

    return carry

  lax.fori_loop(0, ngrp, group, 0)
  plsc.subcore_barrier()

  pltpu.sync_copy(acc.at[pl.ds(sid * RPA, RPA)],
                  out_hbm.at[cid, pl.ds(sid * RPA, RPA)])

  @pl.when(sid == NS - 1)
  def _():
    pltpu.sync_copy(acc.at[pl.ds(NS * RPA, TAIL)],
                    out_hbm.at[cid, pl.ds(NS * RPA, TAIL)])


_B = 1024


def _tc_matmul_body(x_ref, w_ref, o_ref):
  o_ref[...] = jnp.dot(x_ref[...], w_ref[...],
                       preferred_element_type=jnp.float32)


def _tc_matmul(x, w):
  return pl.pallas_call(
      _tc_matmul_body,
      grid=(NPT // _B,),
      in_specs=[
          pl.BlockSpec((_B, D), lambda i: (i, 0)),
          pl.BlockSpec((D, D), lambda i: (0, 0)),
      ],
      out_specs=pl.BlockSpec((_B, D), lambda i: (i, 0)),
      out_shape=jax.ShapeDtypeStruct((NPT, D), jnp.float32),
  )(x, w)


def _tc_scale_body(deg_ref, xw_ref, y_ref, dinv_ref):
  deg = deg_ref[0, :] + deg_ref[1, :] + 1.0
  dinv = lax.rsqrt(deg)
  dinv_ref[...] = dinv
  y_ref[...] = dinv[:, None] * xw_ref[...]


def _tc_scale(deg2, xw):
  return pl.pallas_call(
      _tc_scale_body,
      grid=(NPT // _B,),
      in_specs=[
          pl.BlockSpec((NC, _B), lambda i: (0, i)),
          pl.BlockSpec((_B, D), lambda i: (i, 0)),
      ],
      out_specs=[
          pl.BlockSpec((_B, D), lambda i: (i, 0)),
          pl.BlockSpec((_B,), lambda i: (i,)),
      ],
      out_shape=[
          jax.ShapeDtypeStruct((NPT, D), jnp.float32),
          jax.ShapeDtypeStruct((NPT,), jnp.float32),
      ],
  )(deg2, xw)


def _ln_relu(t, g, b, eps=1e-5):
  m = jnp.mean(t, axis=-1, keepdims=True)
  v = jnp.mean((t - m) * (t - m), axis=-1, keepdims=True)
  h = (t - m) * lax.rsqrt(v + eps) * g[None, :] + b[None, :]
  return jnp.maximum(h, 0.0)


def _tc_mid_body(s_ref, y_ref, dinv_ref, b_ref, g_ref, be_ref, w_ref, o_ref):
  t = s_ref[0] + s_ref[1] + y_ref[...]
  t = dinv_ref[...][:, None] * t + b_ref[...][None, :]
  h = _ln_relu(t, g_ref[...], be_ref[...])
  o_ref[...] = (dinv_ref[...][:, None]
                * jnp.dot(h, w_ref[...], preferred_element_type=jnp.float32))


def _tc_mid(s, y, dinv, b, g, be, w):
  return pl.pallas_call(
      _tc_mid_body,
      grid=(NPT // _B,),
      in_specs=[
          pl.BlockSpec((NC, _B, D), lambda i: (0, i, 0)),
          pl.BlockSpec((_B, D), lambda i: (i, 0)),
          pl.BlockSpec((_B,), lambda i: (i,)),
          pl.BlockSpec((D,), lambda i: (0,)),
          pl.BlockSpec((D,), lambda i: (0,)),
          pl.BlockSpec((D,), lambda i: (0,)),
          pl.BlockSpec((D, D), lambda i: (0, 0)),
      ],
      out_specs=pl.BlockSpec((_B, D), lambda i: (i, 0)),
      out_shape=jax.ShapeDtypeStruct((NPT, D), jnp.float32),
  )(s, y, dinv, b, g, be, w)


def _tc_final_body(s_ref, y_ref, dinv_ref, b_ref, g_ref, be_ref, o_ref):
  t = s_ref[0] + s_ref[1] + y_ref[...]
  t = dinv_ref[...][:, None] * t + b_ref[...][None, :]
  o_ref[...] = _ln_relu(t, g_ref[...], be_ref[...])


def _tc_final(s, y, dinv, b, g, be):
  return pl.pallas_call(
      _tc_final_body,
      grid=(NPT // _B,),
      in_specs=[
          pl.BlockSpec((NC, _B, D), lambda i: (0, i, 0)),
          pl.BlockSpec((_B, D), lambda i: (i, 0)),
          pl.BlockSpec((_B,), lambda i: (i,)),
          pl.BlockSpec((D,), lambda i: (0,)),
          pl.BlockSpec((D,), lambda i: (0,)),
          pl.BlockSpec((D,), lambda i: (0,)),
      ],
      out_specs=pl.BlockSpec((_B, D), lambda i: (i, 0)),
      out_shape=jax.ShapeDtypeStruct((NPT, D), jnp.float32),
  )(s, y, dinv, b, g, be)


def kernel(x, edge_index, W1, b1, g1, be1, W2, b2, g2, be2):
  pad_e = jnp.full((EP - E,), N, dtype=jnp.int32)
  src_p = jnp.concatenate([edge_index[0], pad_e])
  dst_p = jnp.concatenate([edge_index[1], pad_e])
  x_p = jnp.concatenate([x, jnp.zeros((NPT - N, D), x.dtype)], axis=0)

  deg2 = _deg_kernel(dst_p)
  xw1 = _tc_matmul(x_p, W1)
  y1, dinv = _tc_scale(deg2, xw1)
  s1 = _mp_kernel(y1, src_p, dst_p)
  y2 = _tc_mid(s1, y1, dinv, b1, g1, be1, W2)
  s2 = _mp_kernel(y2, src_p, dst_p)
  out = _tc_final(s2, y2, dinv, b2, g2, be2)
  return out[:N]
